# baseline (speedup 1.0000x reference)
import functools

import jax
import jax.numpy as jnp
from jax import lax
from jax.experimental import pallas as pl
from jax.experimental.pallas import tpu as pltpu
from jax.experimental.pallas import tpu_sc as plsc

_BATCH = 16384
_EDIM = 16
_NC = 2
_NS = 16
_NW = _NC * _NS
_BPW = _BATCH // _NW
_GRP = _BPW // 16


def _mf_body(user_ref, item_ref, ut_tbl, vt_tbl, out_ref,
             uidx, iidx, ublk, vblk, outv, sem):
    wid = lax.axis_index("s") * _NC + lax.axis_index("c")
    row0 = pl.multiple_of(wid * _GRP, 8)
    pltpu.sync_copy(user_ref.at[pl.ds(row0, _GRP)], uidx)
    pltpu.sync_copy(item_ref.at[pl.ds(row0, _GRP)], iidx)

    lane = lax.iota(jnp.int32, 16)

    def group(g, _):
        uvec = uidx[g]
        vvec = iidx[g]
        copies = []
        lanes_u, lanes_v, segs_u, segs_v = [], [], [], []
        for l in range(16):
            ru = uvec[l]
            rv = vvec[l]
            offu = lax.div(ru, 128) * 128
            offv = lax.div(rv, 128) * 128
            cu = ru - offu
            cv = rv - offv
            segs_u.append(lax.div(cu, 16) * 16)
            segs_v.append(lax.div(cv, 16) * 16)
            lanes_u.append(lax.rem(cu, 16))
            lanes_v.append(lax.rem(cv, 16))
            copies.append(pltpu.async_copy(
                ut_tbl.at[:, pl.ds(offu, 128)], ublk.at[l], sem))
            copies.append(pltpu.async_copy(
                vt_tbl.at[:, pl.ds(offv, 128)], vblk.at[l], sem))
        for cpy in copies:
            cpy.wait()

        grp = jnp.zeros((16,), jnp.float32)
        for l in range(16):
            splat_u = jnp.full((16,), lanes_u[l], jnp.int32)
            splat_v = jnp.full((16,), lanes_v[l], jnp.int32)
            acc = jnp.zeros((16,), jnp.float32)
            for d in range(_EDIM):
                su = jnp.take(ublk[l, d, pl.ds(segs_u[l], 16)], splat_u)
                sv = jnp.take(vblk[l, d, pl.ds(segs_v[l], 16)], splat_v)
                acc = acc + su * sv
            grp = jnp.where(lane == l, acc, grp)
        outv[g] = 1.0 / (1.0 + jnp.exp(-grp))
        return 0

    lax.fori_loop(0, _GRP, group, 0)
    pltpu.sync_copy(outv, out_ref.at[pl.ds(row0, _GRP)])


_mf_sc = functools.partial(
    pl.kernel,
    out_type=jax.ShapeDtypeStruct((_NW * _GRP, 16), jnp.float32),
    mesh=plsc.VectorSubcoreMesh(
        core_axis_name="c", subcore_axis_name="s",
        num_cores=_NC, num_subcores=_NS),
    scratch_types=[
        pltpu.VMEM((_GRP, 16), jnp.int32),
        pltpu.VMEM((_GRP, 16), jnp.int32),
        pltpu.VMEM((16, _EDIM, 128), jnp.float32),
        pltpu.VMEM((16, _EDIM, 128), jnp.float32),
        pltpu.VMEM((_GRP, 16), jnp.float32),
        pltpu.SemaphoreType.DMA,
    ],
)(_mf_body)


def kernel(user, item, U, V):
    u2 = user.astype(jnp.int32).reshape(_NW * _GRP, 16)
    i2 = item.astype(jnp.int32).reshape(_NW * _GRP, 16)
    out = _mf_sc(u2, i2, U.T, V.T)
    return out.reshape(_BATCH)
